# baseline (speedup 1.0000x reference)
import functools

import jax
import jax.numpy as jnp
from jax import lax
from jax.experimental import pallas as pl
from jax.experimental.pallas import tpu as pltpu
from jax.experimental.pallas import tpu_sc as plsc

NUM_EMB = 100000
NUM_NEW = 128
FEAT = 64
NC, NS, LANES = 2, 16, 16
NW = NC * NS
XROWS, XCOLS = 16384, 50
PADF = 128
HPAD = 56
ROWS_PER_CHUNK = 16
CHUNK = ROWS_PER_CHUNK * XCOLS
GROUPS = CHUNK // LANES
ROWS_PER_W = XROWS // NW
N_CHUNKS = ROWS_PER_W // ROWS_PER_CHUNK
NBUF = 2


def _emb_body(emb, new, idx, out, new_v, idx2d_v, rg_v, cg_v, idxo_v, idxc_v,
              rows_v, gsems, ssems, isems):
    wid = lax.axis_index("s") * NC + lax.axis_index("c")
    row0 = wid * ROWS_PER_W
    pltpu.sync_copy(new, new_v)

    def mk_tables(j, c):
        p = j * LANES + lax.iota(jnp.int32, LANES)
        rg_v[pl.ds(j * LANES, LANES)] = p // XCOLS
        cg_v[pl.ds(j * LANES, LANES)] = p % XCOLS
        return c

    lax.fori_loop(0, GROUPS, mk_tables, 0)
    for b0 in range(NBUF):
        pltpu.async_copy(idx.at[pl.ds(row0 + b0 * ROWS_PER_CHUNK,
                                      ROWS_PER_CHUNK)], idx2d_v[b0], isems[b0])

    def store_chunk(g, b, sem):
        r0 = row0 + g * ROWS_PER_CHUNK
        for k in range(ROWS_PER_CHUNK):
            pltpu.async_copy(
                rows_v[b].at[pl.ds(k * XCOLS, XCOLS), :],
                out.at[pl.ds((r0 + k) * HPAD, XCOLS), pl.ds(0, FEAT)], sem)

    def drain_chunk(b, sem):
        for k in range(ROWS_PER_CHUNK):
            pltpu.make_async_copy(
                rows_v[b].at[pl.ds(k * XCOLS, XCOLS), :],
                out.at[pl.ds(row0 * HPAD, XCOLS), pl.ds(0, FEAT)], sem).wait()

    def idx_slice(g):
        return idx.at[pl.ds(row0 + g * ROWS_PER_CHUNK, ROWS_PER_CHUNK)]

    def prep(g, b):
        pltpu.make_async_copy(idx_slice(g), idx2d_v[b], isems[b]).wait()

        def flatten(j, c):
            s = pl.ds(j * LANES, LANES)
            iv = plsc.load_gather(idx2d_v[b], [rg_v[s], cg_v[s]])
            idxo_v[b][s] = iv
            idxc_v[b][s] = jnp.minimum(iv, NUM_EMB - 1)
            return c

        lax.fori_loop(0, GROUPS, flatten, 0)

        @pl.when(g + NBUF < N_CHUNKS)
        def _():
            pltpu.async_copy(idx_slice(g + NBUF), idx2d_v[b], isems[b])

        @pl.when(g >= NBUF)
        def _():
            drain_chunk(b, ssems[b])

        pltpu.async_copy(emb.at[idxc_v[b]], rows_v[b], gsems[b])

    def fixup(b):
        def fix_group(j, c):
            iv = idxo_v[b][pl.ds(j * LANES, LANES)]
            m = iv >= NUM_EMB
            gmax = jnp.max(iv)

            @pl.when(gmax >= NUM_EMB)
            def _():
                rn = jnp.clip(iv - NUM_EMB, 0, NUM_NEW - 1)
                rowpos = j * LANES + lax.iota(jnp.int32, LANES)

                def fix_col(col, cc):
                    csplat = jnp.full((LANES,), col, jnp.int32)
                    vals = plsc.load_gather(new_v, [rn, csplat])
                    plsc.store_scatter(rows_v[b], [rowpos, csplat], vals,
                                       mask=m)
                    return cc

                lax.fori_loop(0, FEAT, fix_col, 0)

            return c

        lax.fori_loop(0, GROUPS, fix_group, 0)

    def finish(g, b):
        pltpu.make_async_copy(emb.at[idxc_v[b]], rows_v[b], gsems[b]).wait()
        fixup(b)
        store_chunk(g, b, ssems[b])

    def pair(t, carry):
        for b in range(NBUF):
            g = NBUF * t + b

            prep(g, b)

            @pl.when(g >= 1)
            def _():
                finish(g - 1, (b - 1) % NBUF)

        return carry

    lax.fori_loop(0, N_CHUNKS // NBUF, pair, 0)
    last = N_CHUNKS - 1
    lb = last % NBUF
    pltpu.make_async_copy(emb.at[idxc_v[lb]], rows_v[lb], gsems[lb]).wait()
    fixup(lb)
    store_chunk(last, lb, ssems[lb])
    drain_chunk(1 - lb, ssems[1 - lb])
    drain_chunk(lb, ssems[lb])


_emb_kernel = functools.partial(
    pl.kernel,
    out_type=jax.ShapeDtypeStruct((XROWS * HPAD, PADF), jnp.float32),
    mesh=plsc.VectorSubcoreMesh(
        core_axis_name="c", subcore_axis_name="s",
        num_cores=NC, num_subcores=NS,
    ),
    compiler_params=pltpu.CompilerParams(
        use_tc_tiling_on_sc=False, needs_layout_passes=False),
    scratch_types=[
        pltpu.VMEM((NUM_NEW, FEAT), jnp.float32),
        [pltpu.VMEM((ROWS_PER_CHUNK, XCOLS), jnp.int32)] * NBUF,
        pltpu.VMEM((CHUNK,), jnp.int32),
        pltpu.VMEM((CHUNK,), jnp.int32),
        [pltpu.VMEM((CHUNK,), jnp.int32)] * NBUF,
        [pltpu.VMEM((CHUNK,), jnp.int32)] * NBUF,
        [pltpu.VMEM((CHUNK, FEAT), jnp.float32)] * NBUF,
        [pltpu.SemaphoreType.DMA] * NBUF,
        [pltpu.SemaphoreType.DMA] * NBUF,
        [pltpu.SemaphoreType.DMA] * NBUF,
    ],
)(_emb_body)


def kernel(x, embedding, new_embedding):
    out = _emb_kernel(embedding, new_embedding, x.astype(jnp.int32))
    return out.reshape(XROWS, HPAD, PADF)[:, :XCOLS, :FEAT]
